# Initial kernel scaffold; baseline (speedup 1.0000x reference)
import functools

import jax
import jax.numpy as jnp
import numpy as np
from jax import lax
from jax.experimental import pallas as pl
from jax.experimental.pallas import tpu as pltpu
from jax.experimental.pallas import tpu_sc as plsc

N = 10000
E = 320000
D = 128
NC = 2
NS = 16
L = 16
N_TAB = 10016
ROWS_PER_TILE = N_TAB // NS
C = 128
E_PAD = 323584
EPT_CONV = E_PAD // NS
EPT_DEG = E_PAD // (NC * NS)

f32 = jnp.float32
i32 = jnp.int32

_CONSTS = None


def _consts():
    global _CONSTS
    if _CONSTS is None:
        mask = np.asarray(jax.random.bernoulli(jax.random.key(42), 0.9, (N, D)))
        maskscale = np.zeros((N_TAB, D), np.float32)
        maskscale[:N] = np.where(mask, np.float32(1.0) / np.float32(0.9), 0.0)
        eps = np.zeros((N_TAB, D), np.float32)
        eps[:N] = np.asarray(jax.random.normal(jax.random.key(43), (N, D),
                                               dtype=jnp.float32))
        rowvalid = np.zeros((N_TAB, 1), np.float32)
        rowvalid[:N] = 1.0
        zeros8 = np.zeros((N_TAB, 8), np.float32)
        _CONSTS = (maskscale, eps, rowvalid, zeros8)
    return _CONSTS


def _degrees_body(dst_hbm, ew_hbm, z_hbm, out0, out1, acc_sh, dst_v, ew_v,
                  vals_v, sem):
    c = lax.axis_index("c")
    s = lax.axis_index("s")
    nbase = s * ROWS_PER_TILE
    pltpu.sync_copy(z_hbm.at[pl.ds(nbase, ROWS_PER_TILE)],
                    acc_sh.at[pl.ds(nbase, ROWS_PER_TILE)])
    zer = jnp.zeros((L,), f32)
    one = jnp.ones((L,), f32)
    for grp in range(C // L):
        rows = jnp.arange(L, dtype=i32) + grp * L
        plsc.store_scatter(vals_v, [rows, jnp.full((L,), 1, i32)], one)
        for col in range(2, 8):
            plsc.store_scatter(vals_v, [rows, jnp.full((L,), col, i32)], zer)
    plsc.subcore_barrier()

    ebase = (c * NS + s) * EPT_DEG

    def chunk(g, carry):
        off = ebase + g * C
        pltpu.sync_copy(dst_hbm.at[pl.ds(off, C)], dst_v)
        pltpu.sync_copy(ew_hbm.at[pl.ds(off, C)], ew_v)
        for grp in range(C // L):
            rows = jnp.arange(L, dtype=i32) + grp * L
            ewslice = ew_v[pl.ds(grp * L, L)]
            plsc.store_scatter(vals_v, [rows, jnp.full((L,), 0, i32)], ewslice)
        pltpu.sync_copy(vals_v, acc_sh.at[dst_v], add=True)
        return carry

    lax.fori_loop(0, EPT_DEG // C, chunk, 0)
    plsc.subcore_barrier()

    @pl.when(c == 0)
    def _():
        pltpu.sync_copy(acc_sh.at[pl.ds(nbase, ROWS_PER_TILE)],
                        out0.at[pl.ds(nbase, ROWS_PER_TILE)])

    @pl.when(c == 1)
    def _():
        pltpu.sync_copy(acc_sh.at[pl.ds(nbase, ROWS_PER_TILE)],
                        out1.at[pl.ds(nbase, ROWS_PER_TILE)])


_degrees_call = pl.kernel(
    _degrees_body,
    out_type=(jax.ShapeDtypeStruct((N_TAB, 8), f32),
              jax.ShapeDtypeStruct((N_TAB, 8), f32)),
    mesh=plsc.VectorSubcoreMesh(core_axis_name="c", subcore_axis_name="s"),
    scratch_types=[
        pltpu.VMEM_SHARED((N_TAB, 8), f32),
        pltpu.VMEM((C,), i32),
        pltpu.VMEM((C,), f32),
        pltpu.VMEM((C, 8), f32),
        pltpu.SemaphoreType.DMA,
    ],
)


def _make_dualconv(weighted):
    def body(tab0_hbm, tab1_hbm, src_hbm, dst_hbm, *rest):
        if weighted:
            (ew_hbm, out0, out1, acc_sh, src_v, dst_v, ew_v, rows_v, sem) = rest
        else:
            (out0, out1, acc_sh, src_v, dst_v, ew_v, rows_v, sem) = rest
            ew_hbm = None
        c = lax.axis_index("c")
        s = lax.axis_index("s")
        nbase = s * ROWS_PER_TILE

        @pl.when(c == 0)
        def _():
            pltpu.sync_copy(tab0_hbm.at[pl.ds(nbase, ROWS_PER_TILE)],
                            acc_sh.at[pl.ds(nbase, ROWS_PER_TILE)])

        @pl.when(c == 1)
        def _():
            pltpu.sync_copy(tab1_hbm.at[pl.ds(nbase, ROWS_PER_TILE)],
                            acc_sh.at[pl.ds(nbase, ROWS_PER_TILE)])

        plsc.subcore_barrier()
        ebase = s * EPT_CONV

        def chunk(g, carry):
            off = ebase + g * C
            pltpu.sync_copy(src_hbm.at[pl.ds(off, C)], src_v)
            pltpu.sync_copy(dst_hbm.at[pl.ds(off, C)], dst_v)

            @pl.when(c == 0)
            def _():
                pltpu.async_copy(tab0_hbm.at[src_v], rows_v, sem).wait()

            @pl.when(c == 1)
            def _():
                pltpu.async_copy(tab1_hbm.at[src_v], rows_v, sem).wait()

            if weighted:
                @pl.when(c == 0)
                def _():
                    pltpu.sync_copy(ew_hbm.at[pl.ds(off, C)], ew_v)

                    def mult_group(i, cc):
                        ewv = ew_v[pl.ds(i * L, L)]
                        for l in range(L):
                            sv = ewv.at[jnp.full((L,), l, i32)].get(
                                mode="promise_in_bounds")
                            e = i * L + l
                            for j in range(D // L):
                                rows_v[e, pl.ds(j * L, L)] = (
                                    rows_v[e, pl.ds(j * L, L)] * sv)
                        return cc

                    lax.fori_loop(0, C // L, mult_group, 0)

            pltpu.sync_copy(rows_v, acc_sh.at[dst_v], add=True)
            return carry

        lax.fori_loop(0, EPT_CONV // C, chunk, 0)
        plsc.subcore_barrier()

        @pl.when(c == 0)
        def _():
            pltpu.sync_copy(acc_sh.at[pl.ds(nbase, ROWS_PER_TILE)],
                            out0.at[pl.ds(nbase, ROWS_PER_TILE)])

        @pl.when(c == 1)
        def _():
            pltpu.sync_copy(acc_sh.at[pl.ds(nbase, ROWS_PER_TILE)],
                            out1.at[pl.ds(nbase, ROWS_PER_TILE)])

    return pl.kernel(
        body,
        out_type=(jax.ShapeDtypeStruct((N_TAB, D), f32),
                  jax.ShapeDtypeStruct((N_TAB, D), f32)),
        mesh=plsc.VectorSubcoreMesh(core_axis_name="c", subcore_axis_name="s"),
        scratch_types=[
            pltpu.VMEM_SHARED((N_TAB, D), f32),
            pltpu.VMEM((C,), i32),
            pltpu.VMEM((C,), i32),
            pltpu.VMEM((C,), f32),
            pltpu.VMEM((C, D), f32),
            pltpu.SemaphoreType.DMA,
        ],
    )


_dualconv_w = _make_dualconv(True)
_dualconv_u = _make_dualconv(False)


def _tc_b_body(x_ref, we_ref, wp_ref, d0_ref, d1_ref,
               tab1_ref, tab4_ref, dinvw_ref, dinvu_ref):
    degw = d0_ref[:, 0:1] + d1_ref[:, 0:1] + 1.0
    degc = d0_ref[:, 1:2] + d1_ref[:, 1:2] + 1.0
    dinvw = lax.rsqrt(degw)
    dinvu = lax.rsqrt(degc)
    dinvw_ref[...] = dinvw
    dinvu_ref[...] = dinvu
    tab1_ref[...] = dinvw * jnp.dot(x_ref[...], we_ref[...],
                                    preferred_element_type=f32)
    tab4_ref[...] = dinvu * jnp.dot(x_ref[...], wp_ref[...],
                                    preferred_element_type=f32)


def _tc_d_body(acc1_ref, acc4_ref, dinvw_ref, dinvu_ref, benc_ref, bpri_ref,
               ms_ref, pe_ref, wpm_ref, bpm_ref, wps_ref, bps_ref,
               wem_ref, wes_ref,
               tab2_ref, tab3_ref, pm_ref, ps_ref):
    enc_t = jnp.maximum(dinvw_ref[...] * acc1_ref[...] + benc_ref[...], 0.0)
    enc_t = enc_t * ms_ref[...]
    prior = jnp.maximum(dinvu_ref[...] * acc4_ref[...] + bpri_ref[...], 0.0)
    prior = prior + pe_ref[...]
    pm_ref[...] = jnp.dot(prior, wpm_ref[...],
                          preferred_element_type=f32) + bpm_ref[...]
    ps_ref[...] = jax.nn.sigmoid(
        jnp.dot(prior, wps_ref[...], preferred_element_type=f32) + bps_ref[...])
    tab2_ref[...] = dinvu_ref[...] * jnp.dot(enc_t, wem_ref[...],
                                             preferred_element_type=f32)
    tab3_ref[...] = dinvu_ref[...] * jnp.dot(enc_t, wes_ref[...],
                                             preferred_element_type=f32)


def _tc_f_body(acc2_ref, acc3_ref, dinvu_ref, bm_ref, bs_ref, pm_ref, ps_ref,
               eps_ref, rv_ref, kl_ref, cz_ref):
    enc_mean = dinvu_ref[...] * acc2_ref[...] + bm_ref[...]
    enc_std = jax.nn.sigmoid(dinvu_ref[...] * acc3_ref[...] + bs_ref[...])
    cz_ref[...] = eps_ref[...] * enc_std + enc_mean
    ps = ps_ref[...] + 1e-9
    es = enc_std + 1e-9
    kl = (2.0 * jnp.log(ps) - 2.0 * jnp.log(es)
          + (es * es + (enc_mean - pm_ref[...]) ** 2) / (ps * ps) - 1.0)
    kl_ref[0, 0] = jnp.sum(kl * rv_ref[...]) * (0.5 / N)


def kernel(edge_index, x, t, edge_score, total_len, train_len,
           W_enc, b_enc, W_enc_mean, b_enc_mean, W_enc_std, b_enc_std,
           W_prior, b_prior, W_pm, b_pm, W_ps, b_ps):
    maskscale, eps, rowvalid, zeros8 = _consts()

    src = edge_index[0].astype(i32)
    dst = edge_index[1].astype(i32)
    pad_e = E_PAD - E
    src_p = jnp.concatenate([src, jnp.full((pad_e,), N, i32)])
    dst_p = jnp.concatenate([dst, jnp.full((pad_e,), N, i32)])
    ew_p = jnp.concatenate([edge_score.astype(f32), jnp.zeros((pad_e,), f32)])
    x_pad = jnp.concatenate([x, jnp.zeros((N_TAB - N, D), f32)])
    b_enc2 = b_enc.reshape(1, D)
    b_pri2 = b_prior.reshape(1, D)
    b_pm2 = b_pm.reshape(1, D)
    b_ps2 = b_ps.reshape(1, D)
    b_m2 = b_enc_mean.reshape(1, D)
    b_s2 = b_enc_std.reshape(1, D)
    iarr = jnp.arange(D)
    tf = jnp.asarray(t, f32)
    angle = tf / jnp.power(jnp.float32(10000.0),
                           (2.0 * (iarr // 2)).astype(f32) / D)
    pe = jnp.where(iarr % 2 == 0, jnp.sin(angle), jnp.cos(angle))
    pe = pe.astype(f32).reshape(1, D)

    deg0, deg1 = _degrees_call(dst_p, ew_p, zeros8)

    tab1, tab4, dinvw, dinvu = pl.pallas_call(
        _tc_b_body,
        out_shape=(jax.ShapeDtypeStruct((N_TAB, D), f32),
                   jax.ShapeDtypeStruct((N_TAB, D), f32),
                   jax.ShapeDtypeStruct((N_TAB, 1), f32),
                   jax.ShapeDtypeStruct((N_TAB, 1), f32)),
    )(x_pad, W_enc, W_prior, deg0, deg1)

    acc1, acc4 = _dualconv_w(tab1, tab4, src_p, dst_p, ew_p)

    tab2, tab3, pm, ps = pl.pallas_call(
        _tc_d_body,
        out_shape=(jax.ShapeDtypeStruct((N_TAB, D), f32),
                   jax.ShapeDtypeStruct((N_TAB, D), f32),
                   jax.ShapeDtypeStruct((N_TAB, D), f32),
                   jax.ShapeDtypeStruct((N_TAB, D), f32)),
    )(acc1, acc4, dinvw, dinvu, b_enc2, b_pri2, maskscale, pe,
      W_pm, b_pm2, W_ps, b_ps2, W_enc_mean, W_enc_std)

    acc2, acc3 = _dualconv_u(tab2, tab3, src_p, dst_p)

    kl2d, conf_z = pl.pallas_call(
        _tc_f_body,
        out_shape=(jax.ShapeDtypeStruct((1, 1), f32),
                   jax.ShapeDtypeStruct((N_TAB, D), f32)),
    )(acc2, acc3, dinvu, b_m2, b_s2, pm, ps, eps, rowvalid)

    return (kl2d[0, 0], conf_z[:N])


# speedup vs baseline: 10.7127x; 10.7127x over previous
import functools

import jax
import jax.numpy as jnp
import numpy as np
from jax import lax
from jax.experimental import pallas as pl
from jax.experimental.pallas import tpu as pltpu
from jax.experimental.pallas import tpu_sc as plsc

N = 10000
E = 320000
D = 128
NC = 2
NS = 16
L = 16
N_TAB = 10112
ROWS_PER_TILE = N_TAB // NS
C = 128
E_PAD = 323584
EPT_CONV = E_PAD // NS
EPT_DEG = E_PAD // (NC * NS)

f32 = jnp.float32
i32 = jnp.int32

_CONSTS = None


def _consts():
    global _CONSTS
    if _CONSTS is None:
        rowvalid = np.zeros((N_TAB, 1), np.float32)
        rowvalid[:N] = 1.0
        zeros1 = np.zeros((N_TAB,), np.float32)
        _CONSTS = (rowvalid, zeros1)
    return _CONSTS


def _rng_consts():
    mask = jax.random.bernoulli(jax.random.key(42), 1.0 - 0.1, (N, D))
    maskscale = jnp.where(mask, jnp.float32(1.0 / 0.9), jnp.float32(0.0))
    maskscale = jnp.concatenate([maskscale, jnp.zeros((N_TAB - N, D), f32)])
    eps = jax.random.normal(jax.random.key(43), (N, D), dtype=f32)
    eps = jnp.concatenate([eps, jnp.zeros((N_TAB - N, D), f32)])
    return maskscale, eps


def _degrees_body(dst_hbm, ew_hbm, outw0, outc0, outw1, outc1,
                  accw_sh, accc_sh, dst_v, ew_v, ones_v, stage_v, sem):
    c = lax.axis_index("c")
    s = lax.axis_index("s")
    nbase = s * ROWS_PER_TILE
    zer = jnp.zeros((L,), f32)
    for grp in range(640 // L):
        stage_v[pl.ds(grp * L, L)] = zer
    pltpu.sync_copy(stage_v.at[pl.ds(0, ROWS_PER_TILE)],
                    accw_sh.at[pl.ds(nbase, ROWS_PER_TILE)])
    pltpu.sync_copy(stage_v.at[pl.ds(0, ROWS_PER_TILE)],
                    accc_sh.at[pl.ds(nbase, ROWS_PER_TILE)])
    one = jnp.ones((L,), f32)
    for grp in range(C // L):
        ones_v[pl.ds(grp * L, L)] = one
    plsc.subcore_barrier()

    ebase = (c * NS + s) * EPT_DEG

    def chunk(g, carry):
        off = ebase + g * C
        pltpu.sync_copy(dst_hbm.at[pl.ds(off, C)], dst_v)
        pltpu.sync_copy(ew_hbm.at[pl.ds(off, C)], ew_v)
        pltpu.sync_copy(ew_v, accw_sh.at[dst_v], add=True)
        pltpu.sync_copy(ones_v, accc_sh.at[dst_v], add=True)
        return carry

    lax.fori_loop(0, EPT_DEG // C, chunk, 0)
    plsc.subcore_barrier()

    def writeout(acc_sh, out_hbm):
        pltpu.sync_copy(acc_sh.at[pl.ds(nbase, ROWS_PER_TILE)],
                        stage_v.at[pl.ds(0, ROWS_PER_TILE)])
        pltpu.sync_copy(stage_v.at[pl.ds(0, ROWS_PER_TILE)],
                        out_hbm.at[pl.ds(nbase, ROWS_PER_TILE)])

    @pl.when(c == 0)
    def _():
        writeout(accw_sh, outw0)
        writeout(accc_sh, outc0)

    @pl.when(c == 1)
    def _():
        writeout(accw_sh, outw1)
        writeout(accc_sh, outc1)


_degrees_call = pl.kernel(
    _degrees_body,
    out_type=(jax.ShapeDtypeStruct((N_TAB,), f32),
              jax.ShapeDtypeStruct((N_TAB,), f32),
              jax.ShapeDtypeStruct((N_TAB,), f32),
              jax.ShapeDtypeStruct((N_TAB,), f32)),
    mesh=plsc.VectorSubcoreMesh(core_axis_name="c", subcore_axis_name="s"),
    scratch_types=[
        pltpu.VMEM_SHARED((N_TAB,), f32),
        pltpu.VMEM_SHARED((N_TAB,), f32),
        pltpu.VMEM((C,), i32),
        pltpu.VMEM((C,), f32),
        pltpu.VMEM((C,), f32),
        pltpu.VMEM((640,), f32),
        pltpu.SemaphoreType.DMA,
    ],
)


def _make_dualconv(weighted):
    def body(tab0_hbm, tab1_hbm, src_hbm, dst_hbm, *rest):
        if weighted:
            (ew_hbm, out0, out1, acc_sh, src_v, dst_v, ew_v, rows_v, sem) = rest
        else:
            (out0, out1, acc_sh, src_v, dst_v, ew_v, rows_v, sem) = rest
            ew_hbm = None
        c = lax.axis_index("c")
        s = lax.axis_index("s")
        nbase = s * ROWS_PER_TILE

        @pl.when(c == 0)
        def _():
            pltpu.sync_copy(tab0_hbm.at[pl.ds(nbase, ROWS_PER_TILE)],
                            acc_sh.at[pl.ds(nbase, ROWS_PER_TILE)])

        @pl.when(c == 1)
        def _():
            pltpu.sync_copy(tab1_hbm.at[pl.ds(nbase, ROWS_PER_TILE)],
                            acc_sh.at[pl.ds(nbase, ROWS_PER_TILE)])

        plsc.subcore_barrier()
        ebase = s * EPT_CONV

        def chunk(g, carry):
            off = ebase + g * C
            pltpu.sync_copy(src_hbm.at[pl.ds(off, C)], src_v)
            pltpu.sync_copy(dst_hbm.at[pl.ds(off, C)], dst_v)

            @pl.when(c == 0)
            def _():
                pltpu.async_copy(tab0_hbm.at[src_v], rows_v, sem).wait()

            @pl.when(c == 1)
            def _():
                pltpu.async_copy(tab1_hbm.at[src_v], rows_v, sem).wait()

            if weighted:
                @pl.when(c == 0)
                def _():
                    pltpu.sync_copy(ew_hbm.at[pl.ds(off, C)], ew_v)

                    def mult_group(i, cc):
                        ewv = ew_v[pl.ds(i * L, L)]
                        for l in range(L):
                            sv = ewv.at[jnp.full((L,), l, i32)].get(
                                mode="promise_in_bounds")
                            e = i * L + l
                            for j in range(D // L):
                                rows_v[e, pl.ds(j * L, L)] = (
                                    rows_v[e, pl.ds(j * L, L)] * sv)
                        return cc

                    lax.fori_loop(0, C // L, mult_group, 0)

            pltpu.sync_copy(rows_v, acc_sh.at[dst_v], add=True)
            return carry

        lax.fori_loop(0, EPT_CONV // C, chunk, 0)
        plsc.subcore_barrier()

        @pl.when(c == 0)
        def _():
            pltpu.sync_copy(acc_sh.at[pl.ds(nbase, ROWS_PER_TILE)],
                            out0.at[pl.ds(nbase, ROWS_PER_TILE)])

        @pl.when(c == 1)
        def _():
            pltpu.sync_copy(acc_sh.at[pl.ds(nbase, ROWS_PER_TILE)],
                            out1.at[pl.ds(nbase, ROWS_PER_TILE)])

    return pl.kernel(
        body,
        out_type=(jax.ShapeDtypeStruct((N_TAB, D), f32),
                  jax.ShapeDtypeStruct((N_TAB, D), f32)),
        mesh=plsc.VectorSubcoreMesh(core_axis_name="c", subcore_axis_name="s"),
        scratch_types=[
            pltpu.VMEM_SHARED((N_TAB, D), f32),
            pltpu.VMEM((C,), i32),
            pltpu.VMEM((C,), i32),
            pltpu.VMEM((C,), f32),
            pltpu.VMEM((C, D), f32),
            pltpu.SemaphoreType.DMA,
        ],
    )


_dualconv_w = _make_dualconv(True)
_dualconv_u = _make_dualconv(False)


def _tc_b_body(x_ref, we_ref, wp_ref, dw0_ref, dw1_ref, dc0_ref, dc1_ref,
               tab1_ref, tab4_ref, dinvw_ref, dinvu_ref):
    degw = dw0_ref[...] + dw1_ref[...] + 1.0
    degc = dc0_ref[...] + dc1_ref[...] + 1.0
    dinvw = lax.rsqrt(degw)
    dinvu = lax.rsqrt(degc)
    dinvw_ref[...] = dinvw
    dinvu_ref[...] = dinvu
    tab1_ref[...] = dinvw * jnp.dot(x_ref[...], we_ref[...],
                                    preferred_element_type=f32)
    tab4_ref[...] = dinvu * jnp.dot(x_ref[...], wp_ref[...],
                                    preferred_element_type=f32)


def _tc_d_body(acc1_ref, acc4_ref, dinvw_ref, dinvu_ref, benc_ref, bpri_ref,
               ms_ref, pe_ref, wpm_ref, bpm_ref, wps_ref, bps_ref,
               wem_ref, wes_ref,
               tab2_ref, tab3_ref, pm_ref, ps_ref):
    enc_t = jnp.maximum(dinvw_ref[...] * acc1_ref[...] + benc_ref[...], 0.0)
    enc_t = enc_t * ms_ref[...]
    prior = jnp.maximum(dinvu_ref[...] * acc4_ref[...] + bpri_ref[...], 0.0)
    prior = prior + pe_ref[...]
    pm_ref[...] = jnp.dot(prior, wpm_ref[...],
                          preferred_element_type=f32) + bpm_ref[...]
    ps_ref[...] = jax.nn.sigmoid(
        jnp.dot(prior, wps_ref[...], preferred_element_type=f32) + bps_ref[...])
    tab2_ref[...] = dinvu_ref[...] * jnp.dot(enc_t, wem_ref[...],
                                             preferred_element_type=f32)
    tab3_ref[...] = dinvu_ref[...] * jnp.dot(enc_t, wes_ref[...],
                                             preferred_element_type=f32)


def _tc_f_body(acc2_ref, acc3_ref, dinvu_ref, bm_ref, bs_ref, pm_ref, ps_ref,
               eps_ref, rv_ref, kl_ref, cz_ref):
    enc_mean = dinvu_ref[...] * acc2_ref[...] + bm_ref[...]
    enc_std = jax.nn.sigmoid(dinvu_ref[...] * acc3_ref[...] + bs_ref[...])
    cz_ref[...] = eps_ref[...] * enc_std + enc_mean
    ps = ps_ref[...] + 1e-9
    es = enc_std + 1e-9
    kl = (2.0 * jnp.log(ps) - 2.0 * jnp.log(es)
          + (es * es + (enc_mean - pm_ref[...]) ** 2) / (ps * ps) - 1.0)
    kl_ref[0, 0] = jnp.sum(kl * rv_ref[...]) * (0.5 / N)


def kernel(edge_index, x, t, edge_score, total_len, train_len,
           W_enc, b_enc, W_enc_mean, b_enc_mean, W_enc_std, b_enc_std,
           W_prior, b_prior, W_pm, b_pm, W_ps, b_ps):
    rowvalid, zeros1 = _consts()
    maskscale, eps = _rng_consts()

    src = edge_index[0].astype(i32)
    dst = edge_index[1].astype(i32)
    pad_e = E_PAD - E
    src_p = jnp.concatenate([src, jnp.full((pad_e,), N, i32)])
    dst_p = jnp.concatenate([dst, jnp.full((pad_e,), N, i32)])
    ew_p = jnp.concatenate([edge_score.astype(f32), jnp.zeros((pad_e,), f32)])
    x_pad = jnp.concatenate([x, jnp.zeros((N_TAB - N, D), f32)])
    b_enc2 = b_enc.reshape(1, D)
    b_pri2 = b_prior.reshape(1, D)
    b_pm2 = b_pm.reshape(1, D)
    b_ps2 = b_ps.reshape(1, D)
    b_m2 = b_enc_mean.reshape(1, D)
    b_s2 = b_enc_std.reshape(1, D)
    iarr = jnp.arange(D)
    tf = jnp.asarray(t, f32)
    angle = tf / jnp.power(jnp.float32(10000.0),
                           (2.0 * (iarr // 2)).astype(f32) / D)
    pe = jnp.where(iarr % 2 == 0, jnp.sin(angle), jnp.cos(angle))
    pe = pe.astype(f32).reshape(1, D)

    degw0, degc0, degw1, degc1 = _degrees_call(dst_p, ew_p)

    tab1, tab4, dinvw, dinvu = pl.pallas_call(
        _tc_b_body,
        out_shape=(jax.ShapeDtypeStruct((N_TAB, D), f32),
                   jax.ShapeDtypeStruct((N_TAB, D), f32),
                   jax.ShapeDtypeStruct((N_TAB, 1), f32),
                   jax.ShapeDtypeStruct((N_TAB, 1), f32)),
    )(x_pad, W_enc, W_prior, degw0.reshape(N_TAB, 1),
      degw1.reshape(N_TAB, 1), degc0.reshape(N_TAB, 1),
      degc1.reshape(N_TAB, 1))

    acc1, acc4 = _dualconv_w(tab1, tab4, src_p, dst_p, ew_p)

    tab2, tab3, pm, ps = pl.pallas_call(
        _tc_d_body,
        out_shape=(jax.ShapeDtypeStruct((N_TAB, D), f32),
                   jax.ShapeDtypeStruct((N_TAB, D), f32),
                   jax.ShapeDtypeStruct((N_TAB, D), f32),
                   jax.ShapeDtypeStruct((N_TAB, D), f32)),
    )(acc1, acc4, dinvw, dinvu, b_enc2, b_pri2, maskscale, pe,
      W_pm, b_pm2, W_ps, b_ps2, W_enc_mean, W_enc_std)

    acc2, acc3 = _dualconv_u(tab2, tab3, src_p, dst_p)

    kl2d, conf_z = pl.pallas_call(
        _tc_f_body,
        out_shape=(jax.ShapeDtypeStruct((1, 1), f32),
                   jax.ShapeDtypeStruct((N_TAB, D), f32)),
        out_specs=(pl.BlockSpec(memory_space=pltpu.SMEM),
                   pl.BlockSpec(memory_space=pltpu.VMEM)),
    )(acc2, acc3, dinvu, b_m2, b_s2, pm, ps, eps, rowvalid)

    return (kl2d[0, 0], conf_z[:N])
